# baseline (speedup 1.0000x reference)
import functools

import jax
import jax.numpy as jnp
from jax import lax
from jax.experimental import pallas as pl
from jax.experimental.pallas import tpu as pltpu
from jax.experimental.pallas import tpu_sc as plsc

VOCAB_SIZE = 100000
D_MODEL = 128
BATCH = 1024
SEQ = 200

_NW = 32
_N = BATCH * SEQ
_BPW = _N // _NW
_CH = 128
_PG = 2
_GROWS = _PG * _CH
_NG = _BPW // _GROWS
_NSLOT = 3
_K = 2

_mesh = plsc.VectorSubcoreMesh(core_axis_name="c", subcore_axis_name="s")


@functools.partial(
    pl.kernel,
    out_type=jax.ShapeDtypeStruct((_N, D_MODEL), jnp.float32),
    mesh=_mesh,
    scratch_types=(
        [pltpu.VMEM((_NG * _PG, _CH), jnp.int32)]
        + [pltpu.VMEM((_GROWS, D_MODEL), jnp.float32)] * _NSLOT
        + [pltpu.SemaphoreType.DMA] * (2 * _NSLOT)
    ),
)
def _embed_kernel(idx_hbm, table_hbm, out_hbm, idx_v, *bufs_and_sems):
    rows = bufs_and_sems[:_NSLOT]
    sem_in = bufs_and_sems[_NSLOT:2 * _NSLOT]
    sem_out = bufs_and_sems[2 * _NSLOT:]
    nc = _mesh.num_cores
    wid = lax.axis_index("s") * nc + lax.axis_index("c")
    pltpu.sync_copy(idx_hbm.at[wid], idx_v)
    base = wid * _BPW

    def gather(g, b):
        for q in range(_PG):
            pltpu.async_copy(
                table_hbm.at[idx_v.at[g * _PG + q]],
                rows[b].at[pl.ds(q * _CH, _CH)],
                sem_in[b],
            )

    def wait_gather(b):
        pltpu.make_async_copy(table_hbm.at[idx_v.at[0]], rows[b], sem_in[b]).wait()

    def put(g, b):
        pltpu.async_copy(rows[b], out_hbm.at[pl.ds(base + g * _GROWS, _GROWS)], sem_out[b])

    def wait_put(b):
        pltpu.make_async_copy(rows[b], out_hbm.at[pl.ds(base, _GROWS)], sem_out[b]).wait()

    for g in range(_K):
        gather(g, g % _NSLOT)

    def round_(p, carry):
        g0 = p * _NSLOT
        for j in range(_NSLOT):
            g = g0 + j

            @pl.when(g < _NG)
            def _group(g=g, j=j):
                wait_gather(j)
                put(g, j)

                @pl.when(g + _K < _NG)
                def _issue_next():
                    bn = (j + _K) % _NSLOT
                    @pl.when(g >= _NSLOT - _K)
                    def _():
                        wait_put(bn)

                    gather(g + _K, bn)

        return carry

    _nround = -(-_NG // _NSLOT)
    lax.fori_loop(0, _nround, round_, 0)
    for b in range(_NSLOT):
        wait_put(b)


def kernel(inputs, embedding):
    idx3d = inputs.reshape(_NW, _NG * _PG, _CH)
    out = _embed_kernel(idx3d, embedding)
    return out.reshape(BATCH, SEQ, D_MODEL)


# speedup vs baseline: 1.0188x; 1.0188x over previous
import functools

import jax
import jax.numpy as jnp
from jax import lax
from jax.experimental import pallas as pl
from jax.experimental.pallas import tpu as pltpu
from jax.experimental.pallas import tpu_sc as plsc

VOCAB_SIZE = 100000
D_MODEL = 128
BATCH = 1024
SEQ = 200

_NW = 32
_BPW = BATCH // _NW
_NCHUNK = _BPW

_mesh = plsc.VectorSubcoreMesh(core_axis_name="c", subcore_axis_name="s")


_NSLOT = 4
_K = 3


@functools.partial(
    pl.kernel,
    out_type=jax.ShapeDtypeStruct((BATCH * SEQ, D_MODEL), jnp.float32),
    mesh=_mesh,
    scratch_types=(
        [pltpu.VMEM((_BPW, SEQ), jnp.int32)]
        + [pltpu.VMEM((SEQ, D_MODEL), jnp.float32)] * _NSLOT
        + [pltpu.SemaphoreType.DMA] * (2 * _NSLOT)
    ),
)
def _embed_kernel(idx_hbm, table_hbm, out_hbm, idx_v, *bufs_and_sems):
    rows = bufs_and_sems[:_NSLOT]
    sem_in = bufs_and_sems[_NSLOT:2 * _NSLOT]
    sem_out = bufs_and_sems[2 * _NSLOT:]
    nc = _mesh.num_cores
    wid = lax.axis_index("s") * nc + lax.axis_index("c")
    pltpu.sync_copy(idx_hbm.at[pl.ds(wid * _BPW, _BPW)], idx_v)
    base = wid * _BPW * SEQ

    def gather(g, b):
        pltpu.async_copy(table_hbm.at[idx_v.at[g, pl.ds(0, 128)]],
                         rows[b].at[pl.ds(0, 128)], sem_in[b])
        pltpu.async_copy(table_hbm.at[idx_v.at[g, pl.ds(128, SEQ - 128)]],
                         rows[b].at[pl.ds(128, SEQ - 128)], sem_in[b])

    def wait_gather(b):
        pltpu.make_async_copy(table_hbm.at[idx_v.at[0, pl.ds(0, 128)]],
                              rows[b], sem_in[b]).wait()

    def put(g, b):
        pltpu.async_copy(rows[b], out_hbm.at[pl.ds(base + g * SEQ, SEQ)], sem_out[b])

    def wait_put(b):
        pltpu.make_async_copy(rows[b], out_hbm.at[pl.ds(base, SEQ)], sem_out[b]).wait()

    for g in range(_K):
        gather(g, g % _NSLOT)

    def round_(p, carry):
        g0 = p * _NSLOT
        for j in range(_NSLOT):
            g = g0 + j

            @pl.when(g < _NCHUNK)
            def _chunk(g=g, j=j):
                wait_gather(j)
                put(g, j)

                @pl.when(g + _K < _NCHUNK)
                def _issue_next():
                    bn = (j + _K) % _NSLOT
                    @pl.when(g >= _NSLOT - _K)
                    def _():
                        wait_put(bn)

                    gather(g + _K, bn)

        return carry

    _nround = -(-_NCHUNK // _NSLOT)
    lax.fori_loop(0, _nround, round_, 0)
    for b in range(_NSLOT):
        wait_put(b)


def kernel(inputs, embedding):
    out = _embed_kernel(inputs, embedding)
    return out.reshape(BATCH, SEQ, D_MODEL)


# speedup vs baseline: 1.0212x; 1.0024x over previous
import functools

import jax
import jax.numpy as jnp
from jax import lax
from jax.experimental import pallas as pl
from jax.experimental.pallas import tpu as pltpu
from jax.experimental.pallas import tpu_sc as plsc

VOCAB_SIZE = 100000
D_MODEL = 128
BATCH = 1024
SEQ = 200

_NW = 32
_N = BATCH * SEQ
_BPW = _N // _NW
_CH = 128
_NCHUNK = _BPW // _CH

_mesh = plsc.VectorSubcoreMesh(core_axis_name="c", subcore_axis_name="s")


_NSLOT = 7
_K = 6


@functools.partial(
    pl.kernel,
    out_type=jax.ShapeDtypeStruct((_N, D_MODEL), jnp.float32),
    mesh=_mesh,
    scratch_types=(
        [pltpu.VMEM((_NCHUNK, _CH), jnp.int32)]
        + [pltpu.VMEM((_CH, D_MODEL), jnp.float32)] * _NSLOT
        + [pltpu.SemaphoreType.DMA] * (2 * _NSLOT)
    ),
)
def _embed_kernel(idx_hbm, table_hbm, out_hbm, idx_v, *bufs_and_sems):
    rows = bufs_and_sems[:_NSLOT]
    sem_in = bufs_and_sems[_NSLOT:2 * _NSLOT]
    sem_out = bufs_and_sems[2 * _NSLOT:]
    nc = _mesh.num_cores
    wid = lax.axis_index("s") * nc + lax.axis_index("c")
    pltpu.sync_copy(idx_hbm.at[wid], idx_v)
    base = wid * _BPW

    def gather(g, b):
        pltpu.async_copy(table_hbm.at[idx_v.at[g]], rows[b], sem_in[b])

    def wait_gather(b):
        pltpu.make_async_copy(table_hbm.at[idx_v.at[0]], rows[b], sem_in[b]).wait()

    def put(g, b):
        pltpu.async_copy(rows[b], out_hbm.at[pl.ds(base + g * _CH, _CH)], sem_out[b])

    def wait_put(b):
        pltpu.make_async_copy(rows[b], out_hbm.at[pl.ds(base, _CH)], sem_out[b]).wait()

    for g in range(_K):
        gather(g, g % _NSLOT)

    def round_(p, carry):
        g0 = p * _NSLOT
        for j in range(_NSLOT):
            g = g0 + j

            @pl.when(g < _NCHUNK)
            def _chunk(g=g, j=j):
                wait_gather(j)
                put(g, j)

                @pl.when(g + _K < _NCHUNK)
                def _issue_next():
                    bn = (j + _K) % _NSLOT
                    @pl.when(g >= _NSLOT - _K)
                    def _():
                        wait_put(bn)

                    gather(g + _K, bn)

        return carry

    _nround = -(-_NCHUNK // _NSLOT)
    lax.fori_loop(0, _nround, round_, 0)
    for b in range(_NSLOT):
        wait_put(b)


def kernel(inputs, embedding):
    idx3d = inputs.reshape(_NW, _NCHUNK, _CH)
    out = _embed_kernel(idx3d, embedding)
    return out.reshape(BATCH, SEQ, D_MODEL)
